# baseline (speedup 1.0000x reference)
import functools

import jax
import jax.numpy as jnp
from jax import lax
from jax.experimental import pallas as pl
from jax.experimental.pallas import tpu as pltpu
from jax.experimental.pallas import tpu_sc as plsc

BATCH = 16384
EMBED_DIM = 768
NUM_CORES = 2
NUM_SUBCORES = 16
NUM_WORKERS = NUM_CORES * NUM_SUBCORES
ROWS_PER_WORKER = BATCH // NUM_WORKERS
CHUNK = 128
NUM_CHUNKS = ROWS_PER_WORKER // CHUNK


def _make_gather(num_classes):
  mesh = plsc.VectorSubcoreMesh(core_axis_name="c", subcore_axis_name="s")

  @functools.partial(
      pl.kernel,
      mesh=mesh,
      out_type=jax.ShapeDtypeStruct((BATCH, EMBED_DIM), jnp.float32),
      scratch_types=[
          pltpu.VMEM((NUM_CHUNKS, CHUNK), jnp.int32),
          pltpu.VMEM((CHUNK, EMBED_DIM), jnp.float32),
          pltpu.SemaphoreType.DMA,
      ],
  )
  def body(labels_hbm, table_hbm, out_hbm, idx_v, rows_v, sem):
    wid = lax.axis_index("s") * NUM_CORES + lax.axis_index("c")
    base = wid * ROWS_PER_WORKER
    for k in range(NUM_CHUNKS):
      pltpu.sync_copy(labels_hbm.at[pl.ds(base + k * CHUNK, CHUNK)],
                      idx_v.at[k])
    for k in range(NUM_CHUNKS):
      pltpu.async_copy(table_hbm.at[idx_v.at[k]], rows_v, sem).wait()
      pltpu.sync_copy(rows_v, out_hbm.at[pl.ds(base + k * CHUNK, CHUNK)])

  return body


def kernel(labels, table):
  num_classes = table.shape[0]
  fn = _make_gather(num_classes)
  return fn(labels.astype(jnp.int32), table)
